# baseline (speedup 1.0000x reference)
import functools

import jax
import jax.numpy as jnp
from jax import lax
from jax.experimental import pallas as pl
from jax.experimental.pallas import tpu as pltpu
from jax.experimental.pallas import tpu_sc as plsc

N_NODES = 10000
N_EDGES = 320000
D = 128
N_ETYPES = 8

NUM_WORKERS = 32
EPW = N_EDGES // NUM_WORKERS
CHUNK = 80
NUM_CHUNKS = EPW // CHUNK
GROUPS = CHUNK // 16
_EXP_NO_DMA = False

TC_ROWS = 2000


def _tc_premul_body(h_ref, rel_ref, out_ref):
    r = pl.program_id(0)
    out_ref[...] = h_ref[...] * rel_ref[pl.ds(r, 1), :]


def _build_ht(h, rel_weight):
    nb = N_NODES // TC_ROWS
    return pl.pallas_call(
        _tc_premul_body,
        out_shape=jax.ShapeDtypeStruct((N_ETYPES * N_NODES, D), jnp.float32),
        grid=(N_ETYPES, nb),
        in_specs=[
            pl.BlockSpec((TC_ROWS, D), lambda r, b: (b, 0)),
            pl.BlockSpec((N_ETYPES, D), lambda r, b: (0, 0)),
        ],
        out_specs=pl.BlockSpec((TC_ROWS, D), lambda r, b, _nb=nb: (r * _nb + b, 0)),
    )(h, rel_weight)


W32 = D // 2


def _sc_body(ht_hbm, h_hbm, u_hbm, v_hbm, et_hbm, out_hbm,
             idx_u, idx_v, et_v, rows_u, rows_v, out_v,
             sem_u, sem_v):
    cid = lax.axis_index("c")
    sid = lax.axis_index("s")
    wid = sid * 2 + cid
    wbase = wid * EPW

    pltpu.sync_copy(u_hbm.at[pl.ds(wbase, EPW)], idx_u)
    pltpu.sync_copy(v_hbm.at[pl.ds(wbase, EPW)], idx_v)
    pltpu.sync_copy(et_hbm.at[pl.ds(wbase, EPW)], et_v)

    def idx_body(g, carry):
        sl = pl.ds(g * 16, 16)
        idx_u[sl] = et_v[sl] * N_NODES + idx_u[sl]
        return carry

    lax.fori_loop(0, EPW // 16, idx_body, 0)

    def issue(i, b):
        if _EXP_NO_DMA:
            return
        pltpu.async_copy(ht_hbm.at[idx_u.at[pl.ds(i * CHUNK, CHUNK)]],
                         rows_u.at[b], sem_u.at[b])
        pltpu.async_copy(h_hbm.at[idx_v.at[pl.ds(i * CHUNK, CHUNK)]],
                         rows_v.at[b], sem_v.at[b])

    def wait(b):
        if _EXP_NO_DMA:
            return
        dummy = h_hbm.at[pl.ds(0, CHUNK)]
        pltpu.make_async_copy(dummy, rows_u.at[b], sem_u.at[b]).wait()
        pltpu.make_async_copy(dummy, rows_v.at[b], sem_v.at[b]).wait()

    zeros16 = jnp.zeros((16,), jnp.float32)
    zeros16i = jnp.zeros((16,), jnp.int32)

    def compute(i, b):
        @plsc.parallel_loop(0, GROUPS)
        def group_body(g):
            base = i * CHUNK + g * 16
            out_v[pl.ds(base, 16)] = zeros16
            gb = zeros16i + base
            for k in range(16):
                e = g * 16 + k
                acc = None
                for j in range(D // 32):
                    sl = pl.ds(j * 16, 16)
                    pu = plsc.bitcast(rows_u[b, e, sl], jnp.bfloat16)
                    pv = plsc.bitcast(rows_v[b, e, sl], jnp.bfloat16)
                    pr = pu * pv
                    pa, pb = plsc.unpack(pr, format=plsc.PackFormat.INTERLEAVED)
                    part = pa + pb
                    acc = part if acc is None else acc + part
                plsc.addupdate_scatter(out_v, [gb + k], acc)

    issue(0, 0)

    def pair_body(p, carry):
        i = p * 2
        wait(0)
        issue(i + 1, 1)
        compute(i, 0)
        wait(1)

        @pl.when(i + 2 < NUM_CHUNKS)
        def _():
            issue(i + 2, 0)

        compute(i + 1, 1)
        return carry

    lax.fori_loop(0, NUM_CHUNKS // 2, pair_body, 0)
    wait(0)
    compute(NUM_CHUNKS - 1, 0)

    @plsc.parallel_loop(0, EPW // 16)
    def sig_body(g):
        x = out_v[pl.ds(g * 16, 16)]
        out_v[pl.ds(g * 16, 16)] = 1.0 / (1.0 + jnp.exp(-x))
    pltpu.sync_copy(out_v, out_hbm.at[pl.ds(wbase, EPW)])


@jax.jit
def _pack_rows(x):
    xb = x.astype(jnp.bfloat16)
    lo = lax.bitcast_convert_type(xb[:, 0::2], jnp.uint16).astype(jnp.uint32)
    hi = lax.bitcast_convert_type(xb[:, 1::2], jnp.uint16).astype(jnp.uint32)
    p = lax.bitcast_convert_type(lo | (hi << 16), jnp.int32)
    return jnp.concatenate([p, jnp.zeros_like(p)], axis=1)


def _dist_mul_sc(h, u, v, etype, rel_weight):
    ht = _pack_rows(_build_ht(h, rel_weight))
    hb = _pack_rows(h)
    mesh = plsc.VectorSubcoreMesh(core_axis_name="c", subcore_axis_name="s")
    return pl.kernel(
        _sc_body,
        out_type=jax.ShapeDtypeStruct((N_EDGES,), jnp.float32),
        mesh=mesh,
        scratch_types=[
            pltpu.VMEM((EPW,), jnp.int32),
            pltpu.VMEM((EPW,), jnp.int32),
            pltpu.VMEM((EPW,), jnp.int32),
            pltpu.VMEM((2, CHUNK, D), jnp.int32),
            pltpu.VMEM((2, CHUNK, D), jnp.int32),
            pltpu.VMEM((EPW,), jnp.float32),
            pltpu.SemaphoreType.DMA((2,)),
            pltpu.SemaphoreType.DMA((2,)),
        ],
        compiler_params=pltpu.CompilerParams(needs_layout_passes=False),
    )(ht, hb, u, v, etype)


def kernel(h, u, v, etype, rel_weight):
    u = u.astype(jnp.int32)
    v = v.astype(jnp.int32)
    etype = etype.astype(jnp.int32)
    return _dist_mul_sc(h, u, v, etype, rel_weight)


# speedup vs baseline: 3.0734x; 3.0734x over previous
import functools

import jax
import jax.numpy as jnp
from jax import lax
from jax.experimental import pallas as pl
from jax.experimental.pallas import tpu as pltpu
from jax.experimental.pallas import tpu_sc as plsc

N_NODES = 10000
N_EDGES = 320000
D = 128
N_ETYPES = 8

NUM_WORKERS = 32
EPW = N_EDGES // NUM_WORKERS
CHUNK = 80
NUM_CHUNKS = EPW // CHUNK
GROUPS = CHUNK // 16
_EXP_NO_DMA = False

TC_ROWS = 2000


def _tc_premul_body(h_ref, rel_ref, out_ref):
    r = pl.program_id(0)
    out_ref[...] = h_ref[...] * rel_ref[pl.ds(r, 1), :]


def _build_ht(h, rel_weight):
    nb = N_NODES // TC_ROWS
    return pl.pallas_call(
        _tc_premul_body,
        out_shape=jax.ShapeDtypeStruct((N_ETYPES * N_NODES, D), jnp.float32),
        grid=(N_ETYPES, nb),
        in_specs=[
            pl.BlockSpec((TC_ROWS, D), lambda r, b: (b, 0)),
            pl.BlockSpec((N_ETYPES, D), lambda r, b: (0, 0)),
        ],
        out_specs=pl.BlockSpec((TC_ROWS, D), lambda r, b, _nb=nb: (r * _nb + b, 0)),
    )(h, rel_weight)


W32 = D // 2


def _sc_body(ht_hbm, h_hbm, u_hbm, v_hbm, et_hbm, out_hbm,
             idx_u, idx_v, et_v, rows_u, rows_v, out_v,
             sem_u, sem_v):
    cid = lax.axis_index("c")
    sid = lax.axis_index("s")
    wid = sid * 2 + cid
    wbase = wid * EPW

    pltpu.sync_copy(u_hbm.at[pl.ds(wbase, EPW)], idx_u)
    pltpu.sync_copy(v_hbm.at[pl.ds(wbase, EPW)], idx_v)
    pltpu.sync_copy(et_hbm.at[pl.ds(wbase, EPW)], et_v)

    def idx_body(g, carry):
        sl = pl.ds(g * 16, 16)
        idx_u[sl] = et_v[sl] * N_NODES + idx_u[sl]
        return carry

    lax.fori_loop(0, EPW // 16, idx_body, 0)

    def issue(i, b):
        if _EXP_NO_DMA:
            return
        pltpu.async_copy(ht_hbm.at[idx_u.at[pl.ds(i * CHUNK, CHUNK)]],
                         rows_u.at[b], sem_u.at[b])
        pltpu.async_copy(h_hbm.at[idx_v.at[pl.ds(i * CHUNK, CHUNK)]],
                         rows_v.at[b], sem_v.at[b])

    def wait(b):
        if _EXP_NO_DMA:
            return
        dummy = h_hbm.at[pl.ds(0, CHUNK)]
        pltpu.make_async_copy(dummy, rows_u.at[b], sem_u.at[b]).wait()
        pltpu.make_async_copy(dummy, rows_v.at[b], sem_v.at[b]).wait()

    zeros16 = jnp.zeros((16,), jnp.float32)
    zeros16i = jnp.zeros((16,), jnp.int32)

    def compute(i, b):
        @plsc.parallel_loop(0, GROUPS)
        def group_body(g):
            base = i * CHUNK + g * 16
            out_v[pl.ds(base, 16)] = zeros16
            gb = zeros16i + base
            for k in range(16):
                e = g * 16 + k
                acc = None
                for j in range(D // 32):
                    sl = pl.ds(j * 16, 16)
                    pu = plsc.bitcast(rows_u[b, e, sl], jnp.bfloat16)
                    pv = plsc.bitcast(rows_v[b, e, sl], jnp.bfloat16)
                    pr = pu * pv
                    pa, pb = plsc.unpack(pr, format=plsc.PackFormat.INTERLEAVED)
                    part = pa + pb
                    acc = part if acc is None else acc + part
                plsc.addupdate_scatter(out_v, [gb + k], acc)

    issue(0, 0)

    def pair_body(p, carry):
        i = p * 2
        wait(0)
        issue(i + 1, 1)
        compute(i, 0)
        wait(1)

        @pl.when(i + 2 < NUM_CHUNKS)
        def _():
            issue(i + 2, 0)

        compute(i + 1, 1)
        return carry

    lax.fori_loop(0, NUM_CHUNKS // 2, pair_body, 0)
    wait(0)
    compute(NUM_CHUNKS - 1, 0)

    @plsc.parallel_loop(0, EPW // 16)
    def sig_body(g):
        x = out_v[pl.ds(g * 16, 16)]
        out_v[pl.ds(g * 16, 16)] = 1.0 / (1.0 + jnp.exp(-x))
    pltpu.sync_copy(out_v, out_hbm.at[pl.ds(wbase, EPW)])


@jax.jit
def _pack_rows(x):
    bits = lax.bitcast_convert_type(x, jnp.uint32)
    h16 = (bits + 0x7FFF + ((bits >> 16) & 1)) >> 16
    packed = h16[:, :64] | (h16[:, 64:] << 16)
    packed = lax.bitcast_convert_type(packed, jnp.int32)
    return jnp.concatenate([packed, jnp.zeros_like(packed)], axis=1)


def _dist_mul_sc(h, u, v, etype, rel_weight):
    ht = _pack_rows(_build_ht(h, rel_weight))
    hb = _pack_rows(h)
    mesh = plsc.VectorSubcoreMesh(core_axis_name="c", subcore_axis_name="s")
    return pl.kernel(
        _sc_body,
        out_type=jax.ShapeDtypeStruct((N_EDGES,), jnp.float32),
        mesh=mesh,
        scratch_types=[
            pltpu.VMEM((EPW,), jnp.int32),
            pltpu.VMEM((EPW,), jnp.int32),
            pltpu.VMEM((EPW,), jnp.int32),
            pltpu.VMEM((2, CHUNK, D), jnp.int32),
            pltpu.VMEM((2, CHUNK, D), jnp.int32),
            pltpu.VMEM((EPW,), jnp.float32),
            pltpu.SemaphoreType.DMA((2,)),
            pltpu.SemaphoreType.DMA((2,)),
        ],
        compiler_params=pltpu.CompilerParams(needs_layout_passes=False),
    )(ht, hb, u, v, etype)


def kernel(h, u, v, etype, rel_weight):
    u = u.astype(jnp.int32)
    v = v.astype(jnp.int32)
    etype = etype.astype(jnp.int32)
    return _dist_mul_sc(h, u, v, etype, rel_weight)


# speedup vs baseline: 3.9632x; 1.2895x over previous
import functools

import jax
import jax.numpy as jnp
from jax import lax
from jax.experimental import pallas as pl
from jax.experimental.pallas import tpu as pltpu
from jax.experimental.pallas import tpu_sc as plsc

N_NODES = 10000
N_EDGES = 320000
D = 128
N_ETYPES = 8

NUM_WORKERS = 32
EPW = N_EDGES // NUM_WORKERS
CHUNK = 80
NUM_CHUNKS = EPW // CHUNK
GROUPS = CHUNK // 16
_EXP_NO_DMA = False

TC_ROWS = 2000


def _pack_block(x):
    bits = lax.bitcast_convert_type(x, jnp.uint32)
    h16 = (bits + 0x7FFF + ((bits >> 16) & 1)) >> 16
    packed = lax.bitcast_convert_type(
        h16[:, :64] | (h16[:, 64:] << 16), jnp.int32)
    return jnp.concatenate([packed, jnp.zeros_like(packed)], axis=1)


def _tc_premul_body(h_ref, rel_ref, out_ref):
    r = pl.program_id(0)
    out_ref[...] = _pack_block(h_ref[...] * rel_ref[pl.ds(r, 1), :])


def _tc_pack_body(h_ref, out_ref):
    out_ref[...] = _pack_block(h_ref[...])


def _build_ht(h, rel_weight):
    nb = N_NODES // TC_ROWS
    return pl.pallas_call(
        _tc_premul_body,
        out_shape=jax.ShapeDtypeStruct((N_ETYPES * N_NODES, D), jnp.int32),
        grid=(N_ETYPES, nb),
        in_specs=[
            pl.BlockSpec((TC_ROWS, D), lambda r, b: (b, 0)),
            pl.BlockSpec((N_ETYPES, D), lambda r, b: (0, 0)),
        ],
        out_specs=pl.BlockSpec((TC_ROWS, D), lambda r, b, _nb=nb: (r * _nb + b, 0)),
    )(h, rel_weight)


def _build_hb(h):
    nb = N_NODES // TC_ROWS
    return pl.pallas_call(
        _tc_pack_body,
        out_shape=jax.ShapeDtypeStruct((N_NODES, D), jnp.int32),
        grid=(nb,),
        in_specs=[pl.BlockSpec((TC_ROWS, D), lambda b: (b, 0))],
        out_specs=pl.BlockSpec((TC_ROWS, D), lambda b: (b, 0)),
    )(h)


W32 = D // 2


def _sc_body(ht_hbm, h_hbm, u_hbm, v_hbm, et_hbm, out_hbm,
             idx_u, idx_v, et_v, rows_u, rows_v, out_v,
             sem_u, sem_v):
    cid = lax.axis_index("c")
    sid = lax.axis_index("s")
    wid = sid * 2 + cid
    wbase = wid * EPW

    pltpu.sync_copy(u_hbm.at[pl.ds(wbase, EPW)], idx_u)
    pltpu.sync_copy(v_hbm.at[pl.ds(wbase, EPW)], idx_v)
    pltpu.sync_copy(et_hbm.at[pl.ds(wbase, EPW)], et_v)

    def idx_body(g, carry):
        sl = pl.ds(g * 16, 16)
        idx_u[sl] = et_v[sl] * N_NODES + idx_u[sl]
        return carry

    lax.fori_loop(0, EPW // 16, idx_body, 0)

    def issue(i, b):
        if _EXP_NO_DMA:
            return
        pltpu.async_copy(ht_hbm.at[idx_u.at[pl.ds(i * CHUNK, CHUNK)]],
                         rows_u.at[b], sem_u.at[b])
        pltpu.async_copy(h_hbm.at[idx_v.at[pl.ds(i * CHUNK, CHUNK)]],
                         rows_v.at[b], sem_v.at[b])

    def wait(b):
        if _EXP_NO_DMA:
            return
        dummy = h_hbm.at[pl.ds(0, CHUNK)]
        pltpu.make_async_copy(dummy, rows_u.at[b], sem_u.at[b]).wait()
        pltpu.make_async_copy(dummy, rows_v.at[b], sem_v.at[b]).wait()

    zeros16 = jnp.zeros((16,), jnp.float32)
    zeros16i = jnp.zeros((16,), jnp.int32)

    def compute(i, b):
        @plsc.parallel_loop(0, GROUPS)
        def group_body(g):
            base = i * CHUNK + g * 16
            out_v[pl.ds(base, 16)] = zeros16
            gb = zeros16i + base
            for k in range(16):
                e = g * 16 + k
                acc = None
                for j in range(D // 32):
                    sl = pl.ds(j * 16, 16)
                    pu = plsc.bitcast(rows_u[b, e, sl], jnp.bfloat16)
                    pv = plsc.bitcast(rows_v[b, e, sl], jnp.bfloat16)
                    pr = pu * pv
                    pa, pb = plsc.unpack(pr, format=plsc.PackFormat.INTERLEAVED)
                    part = pa + pb
                    acc = part if acc is None else acc + part
                plsc.addupdate_scatter(out_v, [gb + k], acc)

    issue(0, 0)

    def pair_body(p, carry):
        i = p * 2
        wait(0)
        issue(i + 1, 1)
        compute(i, 0)
        wait(1)

        @pl.when(i + 2 < NUM_CHUNKS)
        def _():
            issue(i + 2, 0)

        compute(i + 1, 1)
        return carry

    lax.fori_loop(0, NUM_CHUNKS // 2, pair_body, 0)
    wait(0)
    compute(NUM_CHUNKS - 1, 0)

    @plsc.parallel_loop(0, EPW // 16)
    def sig_body(g):
        x = out_v[pl.ds(g * 16, 16)]
        out_v[pl.ds(g * 16, 16)] = 1.0 / (1.0 + jnp.exp(-x))
    pltpu.sync_copy(out_v, out_hbm.at[pl.ds(wbase, EPW)])


@jax.jit
def _dist_mul_sc(h, u, v, etype, rel_weight):
    ht = _build_ht(h, rel_weight)
    hb = _build_hb(h)
    mesh = plsc.VectorSubcoreMesh(core_axis_name="c", subcore_axis_name="s")
    return pl.kernel(
        _sc_body,
        out_type=jax.ShapeDtypeStruct((N_EDGES,), jnp.float32),
        mesh=mesh,
        scratch_types=[
            pltpu.VMEM((EPW,), jnp.int32),
            pltpu.VMEM((EPW,), jnp.int32),
            pltpu.VMEM((EPW,), jnp.int32),
            pltpu.VMEM((2, CHUNK, D), jnp.int32),
            pltpu.VMEM((2, CHUNK, D), jnp.int32),
            pltpu.VMEM((EPW,), jnp.float32),
            pltpu.SemaphoreType.DMA((2,)),
            pltpu.SemaphoreType.DMA((2,)),
        ],
        compiler_params=pltpu.CompilerParams(needs_layout_passes=False),
    )(ht, hb, u, v, etype)


def kernel(h, u, v, etype, rel_weight):
    u = u.astype(jnp.int32)
    v = v.astype(jnp.int32)
    etype = etype.astype(jnp.int32)
    return _dist_mul_sc(h, u, v, etype, rel_weight)


# speedup vs baseline: 4.0021x; 1.0098x over previous
import functools

import jax
import jax.numpy as jnp
from jax import lax
from jax.experimental import pallas as pl
from jax.experimental.pallas import tpu as pltpu
from jax.experimental.pallas import tpu_sc as plsc

N_NODES = 10000
N_EDGES = 320000
D = 128
N_ETYPES = 8

NUM_WORKERS = 32
EPW = N_EDGES // NUM_WORKERS
CHUNK = 80
NUM_CHUNKS = EPW // CHUNK
GROUPS = CHUNK // 16
_EXP_NO_DMA = False

TC_ROWS = 2000


def _pack_block(x):
    bits = lax.bitcast_convert_type(x, jnp.uint32)
    h16 = (bits + 0x7FFF + ((bits >> 16) & 1)) >> 16
    packed = lax.bitcast_convert_type(
        h16[:, :64] | (h16[:, 64:] << 16), jnp.int32)
    return jnp.concatenate([packed, jnp.zeros_like(packed)], axis=1)


def _tc_premul_body(h_ref, rel_ref, out_ref):
    r = pl.program_id(0)
    out_ref[...] = _pack_block(h_ref[...] * rel_ref[pl.ds(r, 1), :])


def _tc_pack_body(h_ref, out_ref):
    out_ref[...] = _pack_block(h_ref[...])


def _build_ht(h, rel_weight):
    nb = N_NODES // TC_ROWS
    return pl.pallas_call(
        _tc_premul_body,
        out_shape=jax.ShapeDtypeStruct((N_ETYPES * N_NODES, D), jnp.int32),
        grid=(N_ETYPES, nb),
        in_specs=[
            pl.BlockSpec((TC_ROWS, D), lambda r, b: (b, 0)),
            pl.BlockSpec((N_ETYPES, D), lambda r, b: (0, 0)),
        ],
        out_specs=pl.BlockSpec((TC_ROWS, D), lambda r, b, _nb=nb: (r * _nb + b, 0)),
    )(h, rel_weight)


def _build_hb(h):
    nb = N_NODES // TC_ROWS
    return pl.pallas_call(
        _tc_pack_body,
        out_shape=jax.ShapeDtypeStruct((N_NODES, D), jnp.int32),
        grid=(nb,),
        in_specs=[pl.BlockSpec((TC_ROWS, D), lambda b: (b, 0))],
        out_specs=pl.BlockSpec((TC_ROWS, D), lambda b: (b, 0)),
    )(h)


W32 = D // 2


def _sc_body(ht_hbm, h_hbm, u_hbm, v_hbm, et_hbm, out_hbm,
             idx_u, idx_v, et_v, rows_u, rows_v, out_v,
             sem_u, sem_v):
    cid = lax.axis_index("c")
    sid = lax.axis_index("s")
    wid = sid * 2 + cid
    wbase = wid * EPW

    pltpu.sync_copy(u_hbm.at[pl.ds(wbase, EPW)], idx_u)
    pltpu.sync_copy(v_hbm.at[pl.ds(wbase, EPW)], idx_v)
    pltpu.sync_copy(et_hbm.at[pl.ds(wbase, EPW)], et_v)

    def idx_body(g, carry):
        sl = pl.ds(g * 16, 16)
        idx_u[sl] = et_v[sl] * N_NODES + idx_u[sl]
        return carry

    lax.fori_loop(0, EPW // 16, idx_body, 0)

    def issue(i, b):
        if _EXP_NO_DMA:
            return
        pltpu.async_copy(ht_hbm.at[idx_u.at[pl.ds(i * CHUNK, CHUNK)]],
                         rows_u.at[b], sem_u.at[b])
        pltpu.async_copy(h_hbm.at[idx_v.at[pl.ds(i * CHUNK, CHUNK)]],
                         rows_v.at[b], sem_v.at[b])

    def wait(b):
        if _EXP_NO_DMA:
            return
        dummy = h_hbm.at[pl.ds(0, CHUNK)]
        pltpu.make_async_copy(dummy, rows_u.at[b], sem_u.at[b]).wait()
        pltpu.make_async_copy(dummy, rows_v.at[b], sem_v.at[b]).wait()

    zeros16 = jnp.zeros((16,), jnp.float32)
    zeros16i = jnp.zeros((16,), jnp.int32)

    def compute(i, b):
        @plsc.parallel_loop(0, GROUPS)
        def group_body(g):
            base = i * CHUNK + g * 16
            out_v[pl.ds(base, 16)] = zeros16
            gb = zeros16i + base
            for k in range(16):
                e = g * 16 + k
                acc = None
                for j in range(D // 32):
                    sl = pl.ds(j * 16, 16)
                    pu = plsc.bitcast(rows_u[b, e, sl], jnp.bfloat16)
                    pv = plsc.bitcast(rows_v[b, e, sl], jnp.bfloat16)
                    pr = pu * pv
                    acc = pr if acc is None else acc + pr
                pa, pb = plsc.unpack(acc, format=plsc.PackFormat.INTERLEAVED)
                plsc.addupdate_scatter(out_v, [gb + k], pa + pb)

    issue(0, 0)

    def pair_body(p, carry):
        i = p * 2
        wait(0)
        issue(i + 1, 1)
        compute(i, 0)
        wait(1)

        @pl.when(i + 2 < NUM_CHUNKS)
        def _():
            issue(i + 2, 0)

        compute(i + 1, 1)
        return carry

    lax.fori_loop(0, NUM_CHUNKS // 2, pair_body, 0)
    wait(0)
    compute(NUM_CHUNKS - 1, 0)

    @plsc.parallel_loop(0, EPW // 16)
    def sig_body(g):
        x = out_v[pl.ds(g * 16, 16)]
        out_v[pl.ds(g * 16, 16)] = 1.0 / (1.0 + jnp.exp(-x))
    pltpu.sync_copy(out_v, out_hbm.at[pl.ds(wbase, EPW)])


@jax.jit
def _dist_mul_sc(h, u, v, etype, rel_weight):
    ht = _build_ht(h, rel_weight)
    hb = _build_hb(h)
    mesh = plsc.VectorSubcoreMesh(core_axis_name="c", subcore_axis_name="s")
    return pl.kernel(
        _sc_body,
        out_type=jax.ShapeDtypeStruct((N_EDGES,), jnp.float32),
        mesh=mesh,
        scratch_types=[
            pltpu.VMEM((EPW,), jnp.int32),
            pltpu.VMEM((EPW,), jnp.int32),
            pltpu.VMEM((EPW,), jnp.int32),
            pltpu.VMEM((2, CHUNK, D), jnp.int32),
            pltpu.VMEM((2, CHUNK, D), jnp.int32),
            pltpu.VMEM((EPW,), jnp.float32),
            pltpu.SemaphoreType.DMA((2,)),
            pltpu.SemaphoreType.DMA((2,)),
        ],
        compiler_params=pltpu.CompilerParams(needs_layout_passes=False),
    )(ht, hb, u, v, etype)


def kernel(h, u, v, etype, rel_weight):
    u = u.astype(jnp.int32)
    v = v.astype(jnp.int32)
    etype = etype.astype(jnp.int32)
    return _dist_mul_sc(h, u, v, etype, rel_weight)


# speedup vs baseline: 4.0062x; 1.0010x over previous
import functools

import jax
import jax.numpy as jnp
from jax import lax
from jax.experimental import pallas as pl
from jax.experimental.pallas import tpu as pltpu
from jax.experimental.pallas import tpu_sc as plsc

N_NODES = 10000
N_EDGES = 320000
D = 128
N_ETYPES = 8

NUM_WORKERS = 32
EPW = N_EDGES // NUM_WORKERS
CHUNK = 80
NUM_CHUNKS = EPW // CHUNK
GROUPS = CHUNK // 16

TC_ROWS = 2000


def _pack_block(x):
    bits = lax.bitcast_convert_type(x, jnp.uint32)
    h16 = (bits + 0x7FFF + ((bits >> 16) & 1)) >> 16
    packed = lax.bitcast_convert_type(
        h16[:, :64] | (h16[:, 64:] << 16), jnp.int32)
    return jnp.concatenate([packed, jnp.zeros_like(packed)], axis=1)


def _tc_premul_body(h_ref, rel_ref, out_ref):
    r = pl.program_id(0)
    out_ref[...] = _pack_block(h_ref[...] * rel_ref[pl.ds(r, 1), :])


def _build_tables(h, rel_weight):
    nb = N_NODES // TC_ROWS
    relx = jnp.concatenate(
        [rel_weight, jnp.ones((1, D), jnp.float32)], axis=0)
    return pl.pallas_call(
        _tc_premul_body,
        out_shape=jax.ShapeDtypeStruct(((N_ETYPES + 1) * N_NODES, D), jnp.int32),
        grid=(N_ETYPES + 1, nb),
        in_specs=[
            pl.BlockSpec((TC_ROWS, D), lambda r, b: (b, 0)),
            pl.BlockSpec((N_ETYPES + 1, D), lambda r, b: (0, 0)),
        ],
        out_specs=pl.BlockSpec((TC_ROWS, D), lambda r, b, _nb=nb: (r * _nb + b, 0)),
    )(h, relx)


W32 = D // 2


def _sc_body(ht_hbm, u_hbm, v_hbm, et_hbm, out_hbm,
             idx_u, idx_v, et_v, rows_u, rows_v, out_v,
             sem_u, sem_v):
    cid = lax.axis_index("c")
    sid = lax.axis_index("s")
    wid = sid * 2 + cid
    wbase = wid * EPW

    pltpu.sync_copy(u_hbm.at[pl.ds(wbase, EPW)], idx_u)
    pltpu.sync_copy(v_hbm.at[pl.ds(wbase, EPW)], idx_v)
    pltpu.sync_copy(et_hbm.at[pl.ds(wbase, EPW)], et_v)

    def idx_body(g, carry):
        sl = pl.ds(g * 16, 16)
        idx_u[sl] = et_v[sl] * N_NODES + idx_u[sl]
        idx_v[sl] = idx_v[sl] + N_ETYPES * N_NODES
        return carry

    lax.fori_loop(0, EPW // 16, idx_body, 0)

    def issue(i, b):
        pltpu.async_copy(ht_hbm.at[idx_u.at[pl.ds(i * CHUNK, CHUNK)]],
                         rows_u.at[b], sem_u.at[b])
        pltpu.async_copy(ht_hbm.at[idx_v.at[pl.ds(i * CHUNK, CHUNK)]],
                         rows_v.at[b], sem_v.at[b])

    def wait(b):
        dummy = ht_hbm.at[pl.ds(0, CHUNK)]
        pltpu.make_async_copy(dummy, rows_u.at[b], sem_u.at[b]).wait()
        pltpu.make_async_copy(dummy, rows_v.at[b], sem_v.at[b]).wait()

    zeros16 = jnp.zeros((16,), jnp.float32)
    zeros16i = jnp.zeros((16,), jnp.int32)

    def compute(i, b):
        @plsc.parallel_loop(0, GROUPS)
        def group_body(g):
            base = i * CHUNK + g * 16
            out_v[pl.ds(base, 16)] = zeros16
            gb = zeros16i + base
            for k in range(16):
                e = g * 16 + k
                acc = None
                for j in range(D // 32):
                    sl = pl.ds(j * 16, 16)
                    pu = plsc.bitcast(rows_u[b, e, sl], jnp.bfloat16)
                    pv = plsc.bitcast(rows_v[b, e, sl], jnp.bfloat16)
                    pr = pu * pv
                    acc = pr if acc is None else acc + pr
                pa, pb = plsc.unpack(acc, format=plsc.PackFormat.INTERLEAVED)
                plsc.addupdate_scatter(out_v, [gb + k], pa + pb)

    issue(0, 0)

    def pair_body(p, carry):
        i = p * 2
        wait(0)
        issue(i + 1, 1)
        compute(i, 0)
        wait(1)

        @pl.when(i + 2 < NUM_CHUNKS)
        def _():
            issue(i + 2, 0)

        compute(i + 1, 1)
        return carry

    lax.fori_loop(0, NUM_CHUNKS // 2, pair_body, 0)
    wait(0)
    compute(NUM_CHUNKS - 1, 0)

    @plsc.parallel_loop(0, EPW // 16)
    def sig_body(g):
        x = out_v[pl.ds(g * 16, 16)]
        out_v[pl.ds(g * 16, 16)] = 1.0 / (1.0 + jnp.exp(-x))
    pltpu.sync_copy(out_v, out_hbm.at[pl.ds(wbase, EPW)])


@jax.jit
def _dist_mul_sc(h, u, v, etype, rel_weight):
    ht = _build_tables(h, rel_weight)
    mesh = plsc.VectorSubcoreMesh(core_axis_name="c", subcore_axis_name="s")
    return pl.kernel(
        _sc_body,
        out_type=jax.ShapeDtypeStruct((N_EDGES,), jnp.float32),
        mesh=mesh,
        scratch_types=[
            pltpu.VMEM((EPW,), jnp.int32),
            pltpu.VMEM((EPW,), jnp.int32),
            pltpu.VMEM((EPW,), jnp.int32),
            pltpu.VMEM((2, CHUNK, D), jnp.int32),
            pltpu.VMEM((2, CHUNK, D), jnp.int32),
            pltpu.VMEM((EPW,), jnp.float32),
            pltpu.SemaphoreType.DMA((2,)),
            pltpu.SemaphoreType.DMA((2,)),
        ],
        compiler_params=pltpu.CompilerParams(needs_layout_passes=False),
    )(ht, u, v, etype)


def kernel(h, u, v, etype, rel_weight):
    u = u.astype(jnp.int32)
    v = v.astype(jnp.int32)
    etype = etype.astype(jnp.int32)
    return _dist_mul_sc(h, u, v, etype, rel_weight)


# speedup vs baseline: 4.0093x; 1.0008x over previous
import jax
import jax.numpy as jnp
from jax import lax
from jax.experimental import pallas as pl
from jax.experimental.pallas import tpu as pltpu
from jax.experimental.pallas import tpu_sc as plsc

N_NODES = 10000
N_EDGES = 320000
D = 128
N_ETYPES = 8

NUM_WORKERS = 32
EPW = N_EDGES // NUM_WORKERS
CHUNK = 80
NUM_CHUNKS = EPW // CHUNK
GROUPS = CHUNK // 16

TC_ROWS = 2000


def _pack_block(x):
    bits = lax.bitcast_convert_type(x, jnp.uint32)
    h16 = (bits + 0x7FFF + ((bits >> 16) & 1)) >> 16
    packed = lax.bitcast_convert_type(
        h16[:, :64] | (h16[:, 64:] << 16), jnp.int32)
    return jnp.concatenate([packed, jnp.zeros_like(packed)], axis=1)


def _tc_premul_body(h_ref, rel_ref, out_ref):
    r = pl.program_id(0)
    out_ref[...] = _pack_block(h_ref[...] * rel_ref[pl.ds(r, 1), :])


def _build_tables(h, rel_weight):
    nb = N_NODES // TC_ROWS
    relx = jnp.concatenate(
        [rel_weight, jnp.ones((1, D), jnp.float32)], axis=0)
    return pl.pallas_call(
        _tc_premul_body,
        out_shape=jax.ShapeDtypeStruct(((N_ETYPES + 1) * N_NODES, D), jnp.int32),
        grid=(N_ETYPES + 1, nb),
        in_specs=[
            pl.BlockSpec((TC_ROWS, D), lambda r, b: (b, 0)),
            pl.BlockSpec((N_ETYPES + 1, D), lambda r, b: (0, 0)),
        ],
        out_specs=pl.BlockSpec((TC_ROWS, D), lambda r, b, _nb=nb: (r * _nb + b, 0)),
    )(h, relx)


W32 = D // 2


def _sc_body(ht_hbm, u_hbm, v_hbm, et_hbm, out_hbm,
             idx_u, idx_v, et_v, rows_u, rows_v, out_v,
             sem_u, sem_v):
    cid = lax.axis_index("c")
    sid = lax.axis_index("s")
    wid = sid * 2 + cid
    wbase = wid * EPW

    pltpu.sync_copy(u_hbm.at[pl.ds(wbase, EPW)], idx_u)
    pltpu.sync_copy(v_hbm.at[pl.ds(wbase, EPW)], idx_v)
    pltpu.sync_copy(et_hbm.at[pl.ds(wbase, EPW)], et_v)

    def idx_body(g, carry):
        sl = pl.ds(g * 16, 16)
        idx_u[sl] = et_v[sl] * N_NODES + idx_u[sl]
        idx_v[sl] = idx_v[sl] + N_ETYPES * N_NODES
        return carry

    lax.fori_loop(0, EPW // 16, idx_body, 0)

    def issue(i, b):
        pltpu.async_copy(ht_hbm.at[idx_u.at[pl.ds(i * CHUNK, CHUNK)]],
                         rows_u.at[b], sem_u.at[b])
        pltpu.async_copy(ht_hbm.at[idx_v.at[pl.ds(i * CHUNK, CHUNK)]],
                         rows_v.at[b], sem_v.at[b])

    def wait(b):
        dummy = ht_hbm.at[pl.ds(0, CHUNK)]
        pltpu.make_async_copy(dummy, rows_u.at[b], sem_u.at[b]).wait()
        pltpu.make_async_copy(dummy, rows_v.at[b], sem_v.at[b]).wait()

    zeros16 = jnp.zeros((16,), jnp.float32)
    zeros16i = jnp.zeros((16,), jnp.int32)

    def compute(i, b):
        @plsc.parallel_loop(0, GROUPS)
        def group_body(g):
            base = i * CHUNK + g * 16
            out_v[pl.ds(base, 16)] = zeros16
            gb = zeros16i + base
            for k in range(16):
                e = g * 16 + k
                acc = None
                for j in range(D // 32):
                    sl = pl.ds(j * 16, 16)
                    pu = plsc.bitcast(rows_u[b, e, sl], jnp.bfloat16)
                    pv = plsc.bitcast(rows_v[b, e, sl], jnp.bfloat16)
                    pr = pu * pv
                    acc = pr if acc is None else acc + pr
                pa, pb = plsc.unpack(acc, format=plsc.PackFormat.INTERLEAVED)
                plsc.addupdate_scatter(out_v, [gb + k], pa + pb)

    issue(0, 0)

    def pair_body(p, carry):
        i = p * 2
        wait(0)
        issue(i + 1, 1)
        compute(i, 0)
        wait(1)

        @pl.when(i + 2 < NUM_CHUNKS)
        def _():
            issue(i + 2, 0)

        compute(i + 1, 1)
        return carry

    lax.fori_loop(0, NUM_CHUNKS // 2, pair_body, 0)
    wait(0)
    compute(NUM_CHUNKS - 1, 0)

    @plsc.parallel_loop(0, EPW // 16)
    def sig_body(g):
        x = out_v[pl.ds(g * 16, 16)]
        out_v[pl.ds(g * 16, 16)] = 1.0 / (1.0 + jnp.exp(-x))
    pltpu.sync_copy(out_v, out_hbm.at[pl.ds(wbase, EPW)])


@jax.jit
def _dist_mul_sc(h, u, v, etype, rel_weight):
    ht = _build_tables(h, rel_weight)
    mesh = plsc.VectorSubcoreMesh(core_axis_name="c", subcore_axis_name="s")
    return pl.kernel(
        _sc_body,
        out_type=jax.ShapeDtypeStruct((N_EDGES,), jnp.float32),
        mesh=mesh,
        scratch_types=[
            pltpu.VMEM((EPW,), jnp.int32),
            pltpu.VMEM((EPW,), jnp.int32),
            pltpu.VMEM((EPW,), jnp.int32),
            pltpu.VMEM((2, CHUNK, D), jnp.int32),
            pltpu.VMEM((2, CHUNK, D), jnp.int32),
            pltpu.VMEM((EPW,), jnp.float32),
            pltpu.SemaphoreType.DMA((2,)),
            pltpu.SemaphoreType.DMA((2,)),
        ],
        compiler_params=pltpu.CompilerParams(needs_layout_passes=False),
    )(ht, u, v, etype)


def kernel(h, u, v, etype, rel_weight):
    u = u.astype(jnp.int32)
    v = v.astype(jnp.int32)
    etype = etype.astype(jnp.int32)
    return _dist_mul_sc(h, u, v, etype, rel_weight)
